# baseline (speedup 1.0000x reference)
import functools

import jax
import jax.numpy as jnp
from jax import lax
from jax.experimental import pallas as pl
from jax.experimental.pallas import tpu as pltpu
from jax.experimental.pallas import tpu_sc as plsc

N = 10000
E = 320000
H = 128
OUT = 2
L = 4

NC, NS = 2, 16
NW = NC * NS
CB = 80
CBS = 80
E1 = 128000
E2 = E - E1
NZW = 10
RPT = N // NZW
ZR = 40

BN = 2000
BE = 2000
BE2 = 2560
HP = H // 2

_f32 = jnp.float32
_bf16 = jnp.bfloat16


@functools.cache
def _sc_kernels(esz):
    nch = esz // NW // CBS
    epw = nch * CBS
    mesh = plsc.VectorSubcoreMesh(
        core_axis_name="c", subcore_axis_name="s",
        num_cores=NC, num_subcores=NS,
    )

    ntch = esz // NS // CB
    epw16 = esz // NS

    @functools.partial(
        pl.kernel,
        out_type=(
            jax.ShapeDtypeStruct((esz, H), _f32),
            jax.ShapeDtypeStruct((esz, H), _f32),
        ),
        mesh=mesh,
        scratch_types=[
            pltpu.VMEM((ntch, CB), jnp.int32),
            pltpu.VMEM((2, CB, H), _f32),
            pltpu.VMEM_SHARED((N, H), _f32),
            pltpu.SemaphoreType.DMA((2,)),
            pltpu.SemaphoreType.DMA((2,)),
        ],
    )
    def sc_gather(hA, hB, dsti, srci, xiA, xjB, idx, buf, tab, gs, ws):
        c = lax.axis_index("c")
        s = lax.axis_index("s")

        def run(tsrc, isrc, out):
            @pl.when(s < NZW)
            def _():
                pltpu.sync_copy(tsrc.at[pl.ds(s * RPT, RPT)],
                                tab.at[pl.ds(s * RPT, RPT)])

            pltpu.sync_copy(isrc.at[s], idx)
            plsc.subcore_barrier()
            base = s * epw16

            def gath(j, slot):
                pltpu.async_copy(tab.at[idx.at[j]], buf.at[slot],
                                 gs.at[slot])

            def gath_wait(j, slot):
                pltpu.make_async_copy(tab.at[idx.at[j]], buf.at[slot],
                                      gs.at[slot]).wait()

            def wb(j, slot):
                pltpu.async_copy(buf.at[slot],
                                 out.at[pl.ds(base + j * CB, CB)],
                                 ws.at[slot])

            def wb_wait(j, slot):
                pltpu.make_async_copy(buf.at[slot],
                                      out.at[pl.ds(base + j * CB, CB)],
                                      ws.at[slot]).wait()

            gath(0, 0)

            def body(j, carry):
                slot = lax.rem(j, 2)
                nxt = 1 - slot
                gath_wait(j, slot)

                @pl.when(j >= 1)
                def _():
                    wb_wait(j - 1, nxt)

                @pl.when(j + 1 < ntch)
                def _():
                    gath(j + 1, nxt)

                wb(j, slot)
                return carry

            lax.fori_loop(0, ntch, body, 0)
            wb_wait(ntch - 1, lax.rem(ntch - 1, 2))

        @pl.when(c == 0)
        def _():
            run(hA, dsti, xiA)

        @pl.when(c == 1)
        def _():
            run(hB, srci, xjB)

    @functools.partial(
        pl.kernel,
        out_type=jax.ShapeDtypeStruct((NC, N, H), _f32),
        mesh=mesh,
        scratch_types=[
            pltpu.VMEM((nch, CBS), jnp.int32),
            pltpu.VMEM((ZR, H), _f32),
            pltpu.VMEM((2, CBS, H), _f32),
            pltpu.VMEM_SHARED((N, H), _f32),
            pltpu.SemaphoreType.DMA((2,)),
        ],
    )
    def sc_scatter(ue, srci, out, idx, zbuf, buf, aggsh, lsem):
        c = lax.axis_index("c")
        s = lax.axis_index("s")
        wid = s * NC + c
        base = wid * epw
        pltpu.sync_copy(srci.at[wid], idx)

        def zrow(i, carry):
            for k in range(H // 16):
                zbuf[i, pl.ds(k * 16, 16)] = jnp.zeros((16,), _f32)
            return carry

        lax.fori_loop(0, ZR, zrow, 0)

        @pl.when(s < NZW)
        def _():
            for k in range(RPT // ZR):
                pltpu.sync_copy(zbuf, aggsh.at[pl.ds(s * RPT + k * ZR, ZR)])

        plsc.subcore_barrier()

        def load(j, slot):
            pltpu.async_copy(ue.at[pl.ds(base + j * CBS, CBS)], buf.at[slot],
                             lsem.at[slot])

        def load_wait(j, slot):
            pltpu.make_async_copy(ue.at[pl.ds(base + j * CBS, CBS)],
                                  buf.at[slot], lsem.at[slot]).wait()

        load(0, 0)

        def body(j, carry):
            slot = lax.rem(j, 2)
            load_wait(j, slot)

            @pl.when(j + 1 < nch)
            def _():
                load(j + 1, 1 - slot)

            pltpu.sync_copy(buf.at[slot], aggsh.at[idx.at[j]], add=True)
            return carry

        lax.fori_loop(0, nch, body, 0)
        plsc.subcore_barrier()

        @pl.when(s < NZW)
        def _():
            pltpu.sync_copy(aggsh.at[pl.ds(s * RPT, RPT)],
                            out.at[c, pl.ds(s * RPT, RPT)])

    return sc_gather, sc_scatter


def _dot(a, b):
    return jnp.dot(a, b, preferred_element_type=_f32)


def _rne_hi(x):
    r = jax.lax.bitcast_convert_type(x, jnp.uint32)
    lsb = jax.lax.shift_right_logical(r, jnp.uint32(16)) & jnp.uint32(1)
    return (r + jnp.uint32(0x7FFF) + lsb) & jnp.uint32(0xFFFF0000)


def _pack2(v):
    a = _rne_hi(v[:, :HP])
    b = _rne_hi(v[:, HP:])
    return jax.lax.bitcast_convert_type(
        jax.lax.shift_right_logical(a, jnp.uint32(16)) | b, jnp.int32)


def _unpack2(p):
    u = jax.lax.bitcast_convert_type(p, jnp.uint32)
    a = jax.lax.bitcast_convert_type(
        jax.lax.shift_left(u, jnp.uint32(16)), _f32)
    b = jax.lax.bitcast_convert_type(u & jnp.uint32(0xFFFF0000), _f32)
    return a, b


def _ln(u, g, b):
    mu = jnp.mean(u, axis=-1, keepdims=True)
    d = u - mu
    var = jnp.mean(d * d, axis=-1, keepdims=True)
    return d * lax.rsqrt(var + 1e-5) * g + b


def _wspec(shape):
    return pl.BlockSpec(shape, lambda i: tuple(0 for _ in shape))


def _enc_node_body(x, mx, sx, W1, b1, W2, b2, g, bt, Wa, Wb, h, hA, hB):
    xn = (x[...] - mx[...]) / sx[...]
    t = jnp.maximum(_dot(xn, W1[...]) + b1[...], 0.0)
    u = _dot(t, W2[...]) + b2[...]
    hv = _ln(u, g[...], bt[...])
    h[...] = hv
    hA[...] = _dot(hv, Wa[...])
    hB[...] = _dot(hv, Wb[...])


def _enc_node(xp, mx, sx, W1, b1, W2, b2, g, bt, Wa, Wb):
    DP = xp.shape[1]
    return pl.pallas_call(
        _enc_node_body,
        grid=(N // BN,),
        in_specs=[
            pl.BlockSpec((BN, DP), lambda i: (i, 0)),
            _wspec((1, DP)), _wspec((1, DP)),
            _wspec((DP, H)), _wspec((1, H)), _wspec((H, H)), _wspec((1, H)),
            _wspec((1, H)), _wspec((1, H)),
            _wspec((H, H)), _wspec((H, H)),
        ],
        out_specs=[pl.BlockSpec((BN, H), lambda i: (i, 0))] * 3,
        out_shape=[jax.ShapeDtypeStruct((N, H), _f32)] * 3,
    )(xp, mx, sx, W1, b1, W2, b2, g, bt, Wa, Wb)


def _enc_edge_body(eat, me, se, W1, b1, W2, b2, g, bt, out):
    en = (eat[...] - me[...]) / se[...]
    u1 = jax.lax.dot_general(en, W1[...], (((0,), (0,)), ((), ())),
                             preferred_element_type=_f32)
    t = jnp.maximum(u1 + b1[...], 0.0)
    u = _dotb(t, W2[...]) + b2[...]
    out[...] = _ln(u, g[...], bt[...])


def _enc_edge(eat, me, se, W1, b1, W2, b2, g, bt, off, esz):
    DP = eat.shape[0]
    return pl.pallas_call(
        _enc_edge_body,
        grid=(esz // BE2,),
        in_specs=[
            pl.BlockSpec((DP, BE2), lambda i: (0, i + off)),
            _wspec((DP, 1)), _wspec((DP, 1)),
            _wspec((DP, H)), _wspec((1, H)), _wspec((H, H)), _wspec((1, H)),
            _wspec((1, H)), _wspec((1, H)),
        ],
        out_specs=pl.BlockSpec((BE2, H), lambda i: (i, 0)),
        out_shape=jax.ShapeDtypeStruct((esz, H), _f32),
    )(eat, me, se, W1, b1, W2, b2, g, bt)


def _dotb(a, b):
    return jnp.dot(a.astype(_bf16), b.astype(_bf16),
                   preferred_element_type=_f32)


def _edge_mlp_body(xiA, xjB, ea, W1c, b1, W2, b2, g, bt, ue):
    eav = ea[...]
    pre = xiA[...] + xjB[...] + _dotb(eav, W1c[...]) + b1[...]
    t = jnp.maximum(pre, 0.0)
    u = _dotb(t, W2[...]) + b2[...]
    ue[...] = _ln(u, g[...], bt[...]) + eav


def _edge_mlp(xiA, xjB, ea, W1c, b1, W2, b2, g, bt):
    return pl.pallas_call(
        _edge_mlp_body,
        grid=(ea.shape[0] // BE,),
        in_specs=[
            pl.BlockSpec((BE, H), lambda i: (i, 0)),
            pl.BlockSpec((BE, H), lambda i: (i, 0)),
            pl.BlockSpec((BE, H), lambda i: (i, 0)),
            _wspec((H, H)), _wspec((1, H)), _wspec((H, H)), _wspec((1, H)),
            _wspec((1, H)), _wspec((1, H)),
        ],
        out_specs=pl.BlockSpec((BE, H), lambda i: (i, 0)),
        out_shape=jax.ShapeDtypeStruct((ea.shape[0], H), _f32),
    )(xiA, xjB, ea, W1c, b1, W2, b2, g, bt)


def _node_mlp_body(h, pA, pB, Wa, Wb, b1, W2, b2, g, bt, W1a, W1b, hn, hA, hB):
    hv = h[...]
    agg = pA[0] + pA[1] + pB[0] + pB[1]
    pre = _dot(hv, Wa[...]) + _dot(agg, Wb[...]) + b1[...]
    t = jnp.maximum(pre, 0.0)
    u = _dot(t, W2[...]) + b2[...]
    hnv = hv + _ln(u, g[...], bt[...])
    hn[...] = hnv
    hA[...] = _dot(hnv, W1a[...])
    hB[...] = _dot(hnv, W1b[...])


def _node_mlp(h, pA, pB, Wa, Wb, b1, W2, b2, g, bt, W1a, W1b):
    return pl.pallas_call(
        _node_mlp_body,
        grid=(N // BN,),
        in_specs=[
            pl.BlockSpec((BN, H), lambda i: (i, 0)),
            pl.BlockSpec((NC, BN, H), lambda i: (0, i, 0)),
            pl.BlockSpec((NC, BN, H), lambda i: (0, i, 0)),
            _wspec((H, H)), _wspec((H, H)), _wspec((1, H)), _wspec((H, H)),
            _wspec((1, H)), _wspec((1, H)), _wspec((1, H)),
            _wspec((H, H)), _wspec((H, H)),
        ],
        out_specs=[pl.BlockSpec((BN, H), lambda i: (i, 0))] * 3,
        out_shape=[jax.ShapeDtypeStruct((N, H), _f32)] * 3,
    )(h, pA, pB, Wa, Wb, b1, W2, b2, g, bt, W1a, W1b)


def _node_dec_body(h, pA, pB, Wa, Wb, b1, W2, b2, g, bt, dW1, db1, dW2, db2, out):
    hv = h[...]
    agg = pA[0] + pA[1] + pB[0] + pB[1]
    pre = _dot(hv, Wa[...]) + _dot(agg, Wb[...]) + b1[...]
    t = jnp.maximum(pre, 0.0)
    u = _dot(t, W2[...]) + b2[...]
    hnv = hv + _ln(u, g[...], bt[...])
    t2 = jnp.maximum(_dot(hnv, dW1[...]) + db1[...], 0.0)
    out[...] = _dot(t2, dW2[...]) + db2[...]


def _node_dec(h, pA, pB, Wa, Wb, b1, W2, b2, g, bt, dW1, db1, dW2, db2):
    return pl.pallas_call(
        _node_dec_body,
        grid=(N // BN,),
        in_specs=[
            pl.BlockSpec((BN, H), lambda i: (i, 0)),
            pl.BlockSpec((NC, BN, H), lambda i: (0, i, 0)),
            pl.BlockSpec((NC, BN, H), lambda i: (0, i, 0)),
            _wspec((H, H)), _wspec((H, H)), _wspec((1, H)), _wspec((H, H)),
            _wspec((1, H)), _wspec((1, H)), _wspec((1, H)),
            _wspec((H, H)), _wspec((1, H)), _wspec((H, OUT)), _wspec((1, OUT)),
        ],
        out_specs=pl.BlockSpec((BN, OUT), lambda i: (i, 0)),
        out_shape=jax.ShapeDtypeStruct((N, OUT), _f32),
    )(h, pA, pB, Wa, Wb, b1, W2, b2, g, bt, dW1, db1, dW2, db2)


def kernel(x, edge_index, edge_attr, mean_vec_x, std_vec_x, mean_vec_edge,
           std_vec_edge, nW1, nb1, nW2, nb2, ng, nbt, eW1, eb1, eW2, eb2, eg,
           ebt, peW1, peb1, peW2, peb2, peg, pebt, pnW1, pnb1, pnW2, pnb2,
           png, pnbt, dW1, db1, dW2, db2):
    src = edge_index[0].astype(jnp.int32)
    dst = edge_index[1].astype(jnp.int32)
    srcA = src[:E1].reshape(NW, -1, CBS)
    srcB = src[E1:].reshape(NW, -1, CBS)
    srcA16 = src[:E1].reshape(NS, -1, CB)
    dstA16 = dst[:E1].reshape(NS, -1, CB)
    srcB16 = src[E1:].reshape(NS, -1, CB)
    dstB16 = dst[E1:].reshape(NS, -1, CB)

    mx = mean_vec_x.reshape(1, -1)
    sx = std_vec_x.reshape(1, -1)
    me = mean_vec_edge.reshape(-1, 1)
    se = std_vec_edge.reshape(-1, 1)
    eat = edge_attr.T

    r = lambda v: v.reshape(1, -1)
    W1a = peW1[:, :H]
    W1b = peW1[:, H:2 * H]
    W1c = peW1[:, 2 * H:]
    Wa = pnW1[:, :H]
    Wb = pnW1[:, H:]

    h, hA, hB = _enc_node(x, mx, sx, nW1, r(nb1), nW2, r(nb2), r(ng), r(nbt),
                          W1a[0], W1b[0])
    encargs = (me, se, eW1, r(eb1), eW2, r(eb2), r(eg), r(ebt))
    eaA = _enc_edge(eat, *encargs, 0, E1)
    eaB = _enc_edge(eat, *encargs, E1 // BE2, E2)

    gatherA, scatterA = _sc_kernels(E1)
    gatherB, scatterB = _sc_kernels(E2)
    out = None
    for l in range(L):
        ew = (W1c[l], r(peb1[l]), peW2[l], r(peb2[l]), r(peg[l]), r(pebt[l]))
        xiA_A, xjB_A = gatherA(hA, hB, dstA16, srcA16)
        xiA_B, xjB_B = gatherB(hA, hB, dstB16, srcB16)
        ueA = _edge_mlp(xiA_A, xjB_A, eaA, *ew)
        pA = scatterA(ueA, srcA)
        ueB = _edge_mlp(xiA_B, xjB_B, eaB, *ew)
        pB = scatterB(ueB, srcB)
        if l < L - 1:
            h, hA, hB = _node_mlp(h, pA, pB, Wa[l], Wb[l], r(pnb1[l]),
                                  pnW2[l], r(pnb2[l]), r(png[l]), r(pnbt[l]),
                                  W1a[l + 1], W1b[l + 1])
            eaA, eaB = ueA, ueB
        else:
            out = _node_dec(h, pA, pB, Wa[l], Wb[l], r(pnb1[l]), pnW2[l],
                            r(pnb2[l]), r(png[l]), r(pnbt[l]), dW1, r(db1),
                            dW2, r(db2))
    return out


# speedup vs baseline: 1.0320x; 1.0320x over previous
import functools

import jax
import jax.numpy as jnp
from jax import lax
from jax.experimental import pallas as pl
from jax.experimental.pallas import tpu as pltpu
from jax.experimental.pallas import tpu_sc as plsc

N = 10000
E = 320000
H = 128
OUT = 2
L = 4

NC, NS = 2, 16
NW = NC * NS
CB = 80
CBS = 80
E1 = 192000
E2 = E - E1
NZW = 10
RPT = N // NZW
ZR = 40

BN = 2000
BE = 2000
BE2 = 2560
HP = H // 2

_f32 = jnp.float32
_bf16 = jnp.bfloat16


def _cb_for(n):
    return next(c for c in (120, 80, 40) if n % c == 0)


@functools.cache
def _sc_kernels(esz):
    cbs = _cb_for(esz // NW)
    nch = esz // NW // cbs
    epw = nch * cbs
    mesh = plsc.VectorSubcoreMesh(
        core_axis_name="c", subcore_axis_name="s",
        num_cores=NC, num_subcores=NS,
    )

    cb = _cb_for(esz // NS)
    ntch = esz // NS // cb
    epw16 = esz // NS

    @functools.partial(
        pl.kernel,
        out_type=(
            jax.ShapeDtypeStruct((esz, H), _f32),
            jax.ShapeDtypeStruct((esz, H), _f32),
        ),
        mesh=mesh,
        scratch_types=[
            pltpu.VMEM((ntch, cb), jnp.int32),
            pltpu.VMEM((2, cb, H), _f32),
            pltpu.VMEM_SHARED((N, H), _f32),
            pltpu.SemaphoreType.DMA((2,)),
            pltpu.SemaphoreType.DMA((2,)),
        ],
    )
    def sc_gather(hA, hB, dsti, srci, xiA, xjB, idx, buf, tab, gs, ws):
        c = lax.axis_index("c")
        s = lax.axis_index("s")

        def run(tsrc, isrc, out):
            @pl.when(s < NZW)
            def _():
                pltpu.sync_copy(tsrc.at[pl.ds(s * RPT, RPT)],
                                tab.at[pl.ds(s * RPT, RPT)])

            pltpu.sync_copy(isrc.at[s], idx)
            plsc.subcore_barrier()
            base = s * epw16

            def gath(j, slot):
                pltpu.async_copy(tab.at[idx.at[j]], buf.at[slot],
                                 gs.at[slot])

            def gath_wait(j, slot):
                pltpu.make_async_copy(tab.at[idx.at[j]], buf.at[slot],
                                      gs.at[slot]).wait()

            def wb(j, slot):
                pltpu.async_copy(buf.at[slot],
                                 out.at[pl.ds(base + j * cb, cb)],
                                 ws.at[slot])

            def wb_wait(j, slot):
                pltpu.make_async_copy(buf.at[slot],
                                      out.at[pl.ds(base + j * cb, cb)],
                                      ws.at[slot]).wait()

            gath(0, 0)

            def body(j, carry):
                slot = lax.rem(j, 2)
                nxt = 1 - slot
                gath_wait(j, slot)

                @pl.when(j >= 1)
                def _():
                    wb_wait(j - 1, nxt)

                @pl.when(j + 1 < ntch)
                def _():
                    gath(j + 1, nxt)

                wb(j, slot)
                return carry

            lax.fori_loop(0, ntch, body, 0)
            wb_wait(ntch - 1, lax.rem(ntch - 1, 2))

        @pl.when(c == 0)
        def _():
            run(hA, dsti, xiA)

        @pl.when(c == 1)
        def _():
            run(hB, srci, xjB)

    @functools.partial(
        pl.kernel,
        out_type=jax.ShapeDtypeStruct((NC, N, H), _f32),
        mesh=mesh,
        scratch_types=[
            pltpu.VMEM((nch, cbs), jnp.int32),
            pltpu.VMEM((ZR, H), _f32),
            pltpu.VMEM((2, cbs, H), _f32),
            pltpu.VMEM_SHARED((N, H), _f32),
            pltpu.SemaphoreType.DMA((2,)),
        ],
    )
    def sc_scatter(ue, srci, out, idx, zbuf, buf, aggsh, lsem):
        c = lax.axis_index("c")
        s = lax.axis_index("s")
        wid = s * NC + c
        base = wid * epw
        pltpu.sync_copy(srci.at[wid], idx)

        def zrow(i, carry):
            for k in range(H // 16):
                zbuf[i, pl.ds(k * 16, 16)] = jnp.zeros((16,), _f32)
            return carry

        lax.fori_loop(0, ZR, zrow, 0)

        @pl.when(s < NZW)
        def _():
            for k in range(RPT // ZR):
                pltpu.sync_copy(zbuf, aggsh.at[pl.ds(s * RPT + k * ZR, ZR)])

        plsc.subcore_barrier()

        def load(j, slot):
            pltpu.async_copy(ue.at[pl.ds(base + j * cbs, cbs)], buf.at[slot],
                             lsem.at[slot])

        def load_wait(j, slot):
            pltpu.make_async_copy(ue.at[pl.ds(base + j * cbs, cbs)],
                                  buf.at[slot], lsem.at[slot]).wait()

        load(0, 0)

        def body(j, carry):
            slot = lax.rem(j, 2)
            load_wait(j, slot)

            @pl.when(j + 1 < nch)
            def _():
                load(j + 1, 1 - slot)

            pltpu.sync_copy(buf.at[slot], aggsh.at[idx.at[j]], add=True)
            return carry

        lax.fori_loop(0, nch, body, 0)
        plsc.subcore_barrier()

        @pl.when(s < NZW)
        def _():
            pltpu.sync_copy(aggsh.at[pl.ds(s * RPT, RPT)],
                            out.at[c, pl.ds(s * RPT, RPT)])

    return sc_gather, sc_scatter


def _dot(a, b):
    return jnp.dot(a, b, preferred_element_type=_f32)


def _rne_hi(x):
    r = jax.lax.bitcast_convert_type(x, jnp.uint32)
    lsb = jax.lax.shift_right_logical(r, jnp.uint32(16)) & jnp.uint32(1)
    return (r + jnp.uint32(0x7FFF) + lsb) & jnp.uint32(0xFFFF0000)


def _pack2(v):
    a = _rne_hi(v[:, :HP])
    b = _rne_hi(v[:, HP:])
    return jax.lax.bitcast_convert_type(
        jax.lax.shift_right_logical(a, jnp.uint32(16)) | b, jnp.int32)


def _unpack2(p):
    u = jax.lax.bitcast_convert_type(p, jnp.uint32)
    a = jax.lax.bitcast_convert_type(
        jax.lax.shift_left(u, jnp.uint32(16)), _f32)
    b = jax.lax.bitcast_convert_type(u & jnp.uint32(0xFFFF0000), _f32)
    return a, b


def _ln(u, g, b):
    mu = jnp.mean(u, axis=-1, keepdims=True)
    d = u - mu
    var = jnp.mean(d * d, axis=-1, keepdims=True)
    return d * lax.rsqrt(var + 1e-5) * g + b


def _wspec(shape):
    return pl.BlockSpec(shape, lambda i: tuple(0 for _ in shape))


def _enc_node_body(x, mx, sx, W1, b1, W2, b2, g, bt, Wa, Wb, h, hA, hB):
    xn = (x[...] - mx[...]) / sx[...]
    t = jnp.maximum(_dot(xn, W1[...]) + b1[...], 0.0)
    u = _dot(t, W2[...]) + b2[...]
    hv = _ln(u, g[...], bt[...])
    h[...] = hv
    hA[...] = _dot(hv, Wa[...])
    hB[...] = _dot(hv, Wb[...])


def _enc_node(xp, mx, sx, W1, b1, W2, b2, g, bt, Wa, Wb):
    DP = xp.shape[1]
    return pl.pallas_call(
        _enc_node_body,
        grid=(N // BN,),
        in_specs=[
            pl.BlockSpec((BN, DP), lambda i: (i, 0)),
            _wspec((1, DP)), _wspec((1, DP)),
            _wspec((DP, H)), _wspec((1, H)), _wspec((H, H)), _wspec((1, H)),
            _wspec((1, H)), _wspec((1, H)),
            _wspec((H, H)), _wspec((H, H)),
        ],
        out_specs=[pl.BlockSpec((BN, H), lambda i: (i, 0))] * 3,
        out_shape=[jax.ShapeDtypeStruct((N, H), _f32)] * 3,
    )(xp, mx, sx, W1, b1, W2, b2, g, bt, Wa, Wb)


def _enc_edge_body(eat, me, se, W1, b1, W2, b2, g, bt, out):
    en = (eat[...] - me[...]) / se[...]
    u1 = jax.lax.dot_general(en, W1[...], (((0,), (0,)), ((), ())),
                             preferred_element_type=_f32)
    t = jnp.maximum(u1 + b1[...], 0.0)
    u = _dotb(t, W2[...]) + b2[...]
    out[...] = _ln(u, g[...], bt[...])


def _enc_edge(eat, me, se, W1, b1, W2, b2, g, bt, off, esz):
    DP = eat.shape[0]
    return pl.pallas_call(
        _enc_edge_body,
        grid=(esz // BE2,),
        in_specs=[
            pl.BlockSpec((DP, BE2), lambda i: (0, i + off)),
            _wspec((DP, 1)), _wspec((DP, 1)),
            _wspec((DP, H)), _wspec((1, H)), _wspec((H, H)), _wspec((1, H)),
            _wspec((1, H)), _wspec((1, H)),
        ],
        out_specs=pl.BlockSpec((BE2, H), lambda i: (i, 0)),
        out_shape=jax.ShapeDtypeStruct((esz, H), _f32),
    )(eat, me, se, W1, b1, W2, b2, g, bt)


def _dotb(a, b):
    return jnp.dot(a.astype(_bf16), b.astype(_bf16),
                   preferred_element_type=_f32)


def _edge_mlp_body(xiA, xjB, ea, W1c, b1, W2, b2, g, bt, ue):
    eav = ea[...]
    pre = xiA[...] + xjB[...] + _dotb(eav, W1c[...]) + b1[...]
    t = jnp.maximum(pre, 0.0)
    u = _dotb(t, W2[...]) + b2[...]
    ue[...] = _ln(u, g[...], bt[...]) + eav


def _edge_mlp(xiA, xjB, ea, W1c, b1, W2, b2, g, bt):
    return pl.pallas_call(
        _edge_mlp_body,
        grid=(ea.shape[0] // BE,),
        in_specs=[
            pl.BlockSpec((BE, H), lambda i: (i, 0)),
            pl.BlockSpec((BE, H), lambda i: (i, 0)),
            pl.BlockSpec((BE, H), lambda i: (i, 0)),
            _wspec((H, H)), _wspec((1, H)), _wspec((H, H)), _wspec((1, H)),
            _wspec((1, H)), _wspec((1, H)),
        ],
        out_specs=pl.BlockSpec((BE, H), lambda i: (i, 0)),
        out_shape=jax.ShapeDtypeStruct((ea.shape[0], H), _f32),
    )(xiA, xjB, ea, W1c, b1, W2, b2, g, bt)


def _node_mlp_body(h, pA, pB, Wa, Wb, b1, W2, b2, g, bt, W1a, W1b, hn, hA, hB):
    hv = h[...]
    agg = pA[0] + pA[1] + pB[0] + pB[1]
    pre = _dot(hv, Wa[...]) + _dot(agg, Wb[...]) + b1[...]
    t = jnp.maximum(pre, 0.0)
    u = _dot(t, W2[...]) + b2[...]
    hnv = hv + _ln(u, g[...], bt[...])
    hn[...] = hnv
    hA[...] = _dot(hnv, W1a[...])
    hB[...] = _dot(hnv, W1b[...])


def _node_mlp(h, pA, pB, Wa, Wb, b1, W2, b2, g, bt, W1a, W1b):
    return pl.pallas_call(
        _node_mlp_body,
        grid=(N // BN,),
        in_specs=[
            pl.BlockSpec((BN, H), lambda i: (i, 0)),
            pl.BlockSpec((NC, BN, H), lambda i: (0, i, 0)),
            pl.BlockSpec((NC, BN, H), lambda i: (0, i, 0)),
            _wspec((H, H)), _wspec((H, H)), _wspec((1, H)), _wspec((H, H)),
            _wspec((1, H)), _wspec((1, H)), _wspec((1, H)),
            _wspec((H, H)), _wspec((H, H)),
        ],
        out_specs=[pl.BlockSpec((BN, H), lambda i: (i, 0))] * 3,
        out_shape=[jax.ShapeDtypeStruct((N, H), _f32)] * 3,
    )(h, pA, pB, Wa, Wb, b1, W2, b2, g, bt, W1a, W1b)


def _node_dec_body(h, pA, pB, Wa, Wb, b1, W2, b2, g, bt, dW1, db1, dW2, db2, out):
    hv = h[...]
    agg = pA[0] + pA[1] + pB[0] + pB[1]
    pre = _dot(hv, Wa[...]) + _dot(agg, Wb[...]) + b1[...]
    t = jnp.maximum(pre, 0.0)
    u = _dot(t, W2[...]) + b2[...]
    hnv = hv + _ln(u, g[...], bt[...])
    t2 = jnp.maximum(_dot(hnv, dW1[...]) + db1[...], 0.0)
    out[...] = _dot(t2, dW2[...]) + db2[...]


def _node_dec(h, pA, pB, Wa, Wb, b1, W2, b2, g, bt, dW1, db1, dW2, db2):
    return pl.pallas_call(
        _node_dec_body,
        grid=(N // BN,),
        in_specs=[
            pl.BlockSpec((BN, H), lambda i: (i, 0)),
            pl.BlockSpec((NC, BN, H), lambda i: (0, i, 0)),
            pl.BlockSpec((NC, BN, H), lambda i: (0, i, 0)),
            _wspec((H, H)), _wspec((H, H)), _wspec((1, H)), _wspec((H, H)),
            _wspec((1, H)), _wspec((1, H)), _wspec((1, H)),
            _wspec((H, H)), _wspec((1, H)), _wspec((H, OUT)), _wspec((1, OUT)),
        ],
        out_specs=pl.BlockSpec((BN, OUT), lambda i: (i, 0)),
        out_shape=jax.ShapeDtypeStruct((N, OUT), _f32),
    )(h, pA, pB, Wa, Wb, b1, W2, b2, g, bt, dW1, db1, dW2, db2)


def kernel(x, edge_index, edge_attr, mean_vec_x, std_vec_x, mean_vec_edge,
           std_vec_edge, nW1, nb1, nW2, nb2, ng, nbt, eW1, eb1, eW2, eb2, eg,
           ebt, peW1, peb1, peW2, peb2, peg, pebt, pnW1, pnb1, pnW2, pnb2,
           png, pnbt, dW1, db1, dW2, db2):
    src = edge_index[0].astype(jnp.int32)
    dst = edge_index[1].astype(jnp.int32)
    srcA = src[:E1].reshape(NW, -1, _cb_for(E1 // NW))
    srcB = src[E1:].reshape(NW, -1, _cb_for(E2 // NW))
    srcA16 = src[:E1].reshape(NS, -1, _cb_for(E1 // NS))
    dstA16 = dst[:E1].reshape(NS, -1, _cb_for(E1 // NS))
    srcB16 = src[E1:].reshape(NS, -1, _cb_for(E2 // NS))
    dstB16 = dst[E1:].reshape(NS, -1, _cb_for(E2 // NS))

    mx = mean_vec_x.reshape(1, -1)
    sx = std_vec_x.reshape(1, -1)
    me = mean_vec_edge.reshape(-1, 1)
    se = std_vec_edge.reshape(-1, 1)
    eat = edge_attr.T

    r = lambda v: v.reshape(1, -1)
    W1a = peW1[:, :H]
    W1b = peW1[:, H:2 * H]
    W1c = peW1[:, 2 * H:]
    Wa = pnW1[:, :H]
    Wb = pnW1[:, H:]

    h, hA, hB = _enc_node(x, mx, sx, nW1, r(nb1), nW2, r(nb2), r(ng), r(nbt),
                          W1a[0], W1b[0])
    encargs = (me, se, eW1, r(eb1), eW2, r(eb2), r(eg), r(ebt))
    eaA = _enc_edge(eat, *encargs, 0, E1)
    eaB = _enc_edge(eat, *encargs, E1 // BE2, E2)

    gatherA, scatterA = _sc_kernels(E1)
    gatherB, scatterB = _sc_kernels(E2)
    out = None
    for l in range(L):
        ew = (W1c[l], r(peb1[l]), peW2[l], r(peb2[l]), r(peg[l]), r(pebt[l]))
        xiA_A, xjB_A = gatherA(hA, hB, dstA16, srcA16)
        xiA_B, xjB_B = gatherB(hA, hB, dstB16, srcB16)
        ueA = _edge_mlp(xiA_A, xjB_A, eaA, *ew)
        pA = scatterA(ueA, srcA)
        ueB = _edge_mlp(xiA_B, xjB_B, eaB, *ew)
        pB = scatterB(ueB, srcB)
        if l < L - 1:
            h, hA, hB = _node_mlp(h, pA, pB, Wa[l], Wb[l], r(pnb1[l]),
                                  pnW2[l], r(pnb2[l]), r(png[l]), r(pnbt[l]),
                                  W1a[l + 1], W1b[l + 1])
            eaA, eaB = ueA, ueB
        else:
            out = _node_dec(h, pA, pB, Wa[l], Wb[l], r(pnb1[l]), pnW2[l],
                            r(pnb2[l]), r(png[l]), r(pnbt[l]), dW1, r(db1),
                            dW2, r(db2))
    return out
